# Initial kernel scaffold; baseline (speedup 1.0000x reference)
import jax
import jax.numpy as jnp
from jax.experimental import pallas as pl


def kernel(h, vertices, hadj, pretrained_emb, w_gat, a_src, a_trg, W1, W2, M, Wfc, bfc):
    raise NotImplementedError("write your pallas kernel here")


# speedup vs baseline: 22.9269x; 22.9269x over previous
import jax
import jax.numpy as jnp
from jax import lax
from jax.experimental import pallas as pl
from jax.experimental.pallas import tpu as pltpu
from jax.experimental.pallas import tpu_sc as plsc

N = 10000
N_USER = 8192
E = 320000
D = 192
H = 3
F = 64
K = 2

NC = 2
NS = 16
L = 16

CH = 80
EDGES_PER_TILE = E // NS
NCHUNK = EDGES_PER_TILE // CH

ZCH = 80
NZCHUNK = (N + ZCH - 1) // ZCH

NORM_PER_TILE = N_USER // NS
NCH = 64
NNORM = NORM_PER_TILE // NCH

def _sc_mesh():
    return plsc.VectorSubcoreMesh(
        core_axis_name="c", subcore_axis_name="s",
        num_cores=NC, num_subcores=NS)


def _emb_gather_body(pe_hbm, vidx_hbm, out_hbm, idx_v, rows_v, sem):
    c = lax.axis_index("c")
    s = lax.axis_index("s")
    wid = s * NC + c
    pltpu.sync_copy(vidx_hbm.at[pl.ds(wid * 2, 2)], idx_v)
    for j in range(2):
        pltpu.async_copy(pe_hbm.at[idx_v.at[j]], rows_v.at[j], sem).wait()
        pltpu.sync_copy(rows_v.at[j], out_hbm.at[pl.ds(wid * 256 + j * 128, 128)])


def _emb_gather(pe, vidx):
    return pl.kernel(
        _emb_gather_body,
        out_type=jax.ShapeDtypeStruct((N_USER, F), jnp.float32),
        mesh=_sc_mesh(),
        compiler_params=pltpu.CompilerParams(use_tc_tiling_on_sc=False, needs_layout_passes=False),
        scratch_types=[
            pltpu.VMEM((2, 128), jnp.int32),
            pltpu.VMEM((2, 128, F), jnp.float32),
            pltpu.SemaphoreType.DMA,
        ],
    )(pe, vidx)


def _proj_body(hf_ref, w_ref, as_ref, at_ref, hp_ref, asrc_ref, atrg_ref):
    x = hf_ref[...]
    hp = jnp.dot(x, w_ref[0], preferred_element_type=jnp.float32)
    hp_ref[0] = hp
    asrc_ref[0] = jnp.dot(hp, as_ref[0], preferred_element_type=jnp.float32)[:, :H]
    atrg_ref[0] = jnp.dot(hp, at_ref[0], preferred_element_type=jnp.float32)[:, :H]


def _tc_proj(hf, w_cat, A_s, A_t):
    BN = 2000
    return pl.pallas_call(
        _proj_body,
        grid=(K, N // BN),
        in_specs=[
            pl.BlockSpec((BN, D), lambda k, b: (b, 0)),
            pl.BlockSpec((1, D, H * F), lambda k, b: (k, 0, 0)),
            pl.BlockSpec((1, H * F, 8), lambda k, b: (k, 0, 0)),
            pl.BlockSpec((1, H * F, 8), lambda k, b: (k, 0, 0)),
        ],
        out_specs=[
            pl.BlockSpec((1, BN, H * F), lambda k, b: (k, b, 0)),
            pl.BlockSpec((1, BN, H), lambda k, b: (k, b, 0)),
            pl.BlockSpec((1, BN, H), lambda k, b: (k, b, 0)),
        ],
        out_shape=[
            jax.ShapeDtypeStruct((K, N, H * F), jnp.float32),
            jax.ShapeDtypeStruct((K, N, H), jnp.float32),
            jax.ShapeDtypeStruct((K, N, H), jnp.float32),
        ],
    )(hf, w_cat, A_s, A_t)


def _edge_body(hph_hbm, asrc_hbm, atrg_hbm, src_hbm, trg_hbm, x_hbm,
               asrc_v, atrg_v, sidx_v, tidx_v, upd_v, w_v, rows_v, out_v, sem,
               num_sp, den_sp):
    c = lax.axis_index("c")
    s = lax.axis_index("s")

    iota = lax.iota(jnp.int32, L)
    zeros16 = jnp.zeros((L,), jnp.float32)
    rhalf = iota // 8
    chalf = iota % 8
    iota8 = iota * 8
    base_e = s * EDGES_PER_TILE
    cN = c * N
    nbase = s * NORM_PER_TILE

    for h in range(H):
        def _z_rows(j, carry):
            for cg in range(F // L):
                rows_v[j, pl.ds(cg * L, L)] = zeros16
            return carry
        lax.fori_loop(0, ZCH, _z_rows, None)

        def _z_upd(g, carry):
            plsc.store_scatter(upd_v, [g * 2 + rhalf, chalf], zeros16)
            return carry
        lax.fori_loop(0, CH * 8 // L, _z_upd, None)

        plsc.subcore_barrier()

        for i in range(8):
            m = s + i * NS

            @pl.when(m < NZCHUNK)
            def _():
                pltpu.sync_copy(rows_v, num_sp.at[pl.ds(m * ZCH, ZCH)])
                if h == 0:
                    pltpu.sync_copy(upd_v, den_sp.at[pl.ds(m * ZCH, ZCH)])

        pltpu.sync_copy(asrc_hbm.at[c * H + h], asrc_v)
        pltpu.sync_copy(atrg_hbm.at[c * H + h], atrg_v)

        plsc.subcore_barrier()

        def _chunk(cc, carry):
            off = base_e + cc * CH
            pltpu.sync_copy(src_hbm.at[c, pl.ds(off, CH)], sidx_v.at[0])
            pltpu.sync_copy(trg_hbm.at[c, pl.ds(off, CH)], tidx_v.at[0])

            for g in range(CH // L):
                s16 = sidx_v[0, pl.ds(g * L, L)]
                t16 = tidx_v[0, pl.ds(g * L, L)]
                a_s = plsc.load_gather(asrc_v, [s16])
                a_t = plsc.load_gather(atrg_v, [t16])
                e = a_s + a_t
                e = jnp.where(e > 0.0, e, 0.2 * e)
                x = jnp.exp(e)
                r16 = iota + g * L
                plsc.store_scatter(upd_v, [r16, jnp.full((L,), h, jnp.int32)], x)
                plsc.store_scatter(w_v, [iota8 + g * L * 8], x)
                sidx_v[0, pl.ds(g * L, L)] = s16 + cN

            pltpu.async_copy(hph_hbm.at[h].at[sidx_v.at[0]], rows_v, sem).wait()

            def _scale(j, carry2):
                wvec = w_v[pl.ds(j * 8, L)]
                w0 = wvec[0]
                for cg in range(F // L):
                    rows_v[j, pl.ds(cg * L, L)] = rows_v[j, pl.ds(cg * L, L)] * w0
                return carry2
            lax.fori_loop(0, CH, _scale, None)

            pltpu.sync_copy(rows_v, num_sp.at[tidx_v.at[0]], add=True)
            pltpu.sync_copy(upd_v, den_sp.at[tidx_v.at[0]], add=True)
            return carry

        lax.fori_loop(0, NCHUNK, _chunk, None)

        plsc.subcore_barrier()

        def _norm(ncc, carry):
            nb = nbase + ncc * NCH
            pltpu.sync_copy(num_sp.at[pl.ds(nb, NCH)], rows_v.at[pl.ds(0, NCH)])
            pltpu.sync_copy(den_sp.at[pl.ds(nb, NCH)], upd_v.at[pl.ds(0, NCH)])

            for jg in range(NCH // L):
                n16 = iota + jg * L
                d16 = plsc.load_gather(upd_v, [n16, jnp.full((L,), h, jnp.int32)])
                r16 = 1.0 / (d16 + 1e-16)
                plsc.store_scatter(w_v, [iota8 + jg * L * 8], r16)

            def _node(j, carry2):
                rvec = w_v[pl.ds(j * 8, L)]
                r0 = rvec[0]
                for cg in range(F // L):
                    out_v[j, pl.ds(cg * L, L)] = rows_v[j, pl.ds(cg * L, L)] * r0
                return carry2
            lax.fori_loop(0, NCH, _node, None)

            pltpu.sync_copy(out_v, x_hbm.at[c * H + h, pl.ds(nb, NCH)])
            return carry
        lax.fori_loop(0, NNORM, _norm, None)


def _sc_edge(hph, asrc_t, atrg_t, src2, trg2):
    return pl.kernel(
        _edge_body,
        out_type=jax.ShapeDtypeStruct((K * H, N_USER, F), jnp.float32),
        mesh=_sc_mesh(),
        compiler_params=pltpu.CompilerParams(use_tc_tiling_on_sc=False, needs_layout_passes=False),
        scratch_types=[
            pltpu.VMEM((N,), jnp.float32),
            pltpu.VMEM((N,), jnp.float32),
            pltpu.VMEM((1, CH), jnp.int32),
            pltpu.VMEM((1, CH), jnp.int32),
            pltpu.VMEM((CH, 8), jnp.float32),
            pltpu.VMEM((CH * 8 + L,), jnp.float32),
            pltpu.VMEM((CH, F), jnp.float32),
            pltpu.VMEM((NCH, F), jnp.float32),
            pltpu.SemaphoreType.DMA,
            pltpu.VMEM_SHARED((N, F), jnp.float32),
            pltpu.VMEM_SHARED((N, 8), jnp.float32),
        ],
    )(hph, asrc_t, atrg_t, src2, trg2)


def _fusion_body(hf_ref, x_ref, w1_ref, w2_ref, m_ref, wfc_ref, bfc_ref, out_ref):
    f = hf_ref[...]
    third = jnp.float32(1.0 / 3.0)
    x0 = (x_ref[0, 0] + x_ref[0, 1] + x_ref[0, 2]) * third
    x1 = (x_ref[1, 0] + x_ref[1, 1] + x_ref[1, 2]) * third
    fw1 = jnp.dot(f, w1_ref[...], preferred_element_type=jnp.float32)
    q0 = jnp.tanh(fw1 + jnp.dot(x0, w2_ref[...], preferred_element_type=jnp.float32))
    q1 = jnp.tanh(fw1 + jnp.dot(x1, w2_ref[...], preferred_element_type=jnp.float32))
    s0 = jnp.dot(q0, m_ref[...], preferred_element_type=jnp.float32)
    s1 = jnp.dot(q1, m_ref[...], preferred_element_type=jnp.float32)
    m = jnp.maximum(s0, s1)
    e0 = jnp.exp(s0 - m)
    e1 = jnp.exp(s1 - m)
    denom = e0 + e1
    b0 = e0 / denom
    b1 = e1 / denom
    fus = b0 * x0 + b1 * x1
    wfc = wfc_ref[...]
    logits = (jnp.dot(x0, wfc[0:F], preferred_element_type=jnp.float32)
              + jnp.dot(x1, wfc[F:2 * F], preferred_element_type=jnp.float32)
              + jnp.dot(fus, wfc[2 * F:3 * F], preferred_element_type=jnp.float32)
              + bfc_ref[...])
    lm = jnp.max(logits, axis=-1, keepdims=True)
    lse = lm + jnp.log(jnp.sum(jnp.exp(logits - lm), axis=-1, keepdims=True))
    out_ref[...] = logits - lse


def _tc_fusion(hf, x, W1, W2, M, Wfc, bfc2):
    BN = 512
    return pl.pallas_call(
        _fusion_body,
        grid=(N_USER // BN,),
        in_specs=[
            pl.BlockSpec((BN, D), lambda b: (b, 0)),
            pl.BlockSpec((K, H, BN, F), lambda b: (0, 0, b, 0)),
            pl.BlockSpec((D, F), lambda b: (0, 0)),
            pl.BlockSpec((F, F), lambda b: (0, 0)),
            pl.BlockSpec((F, 1), lambda b: (0, 0)),
            pl.BlockSpec((H * F, K), lambda b: (0, 0)),
            pl.BlockSpec((1, K), lambda b: (0, 0)),
        ],
        out_specs=pl.BlockSpec((BN, K), lambda b: (b, 0)),
        out_shape=jax.ShapeDtypeStruct((N_USER, K), jnp.float32),
    )(hf, x, W1, W2, M, Wfc, bfc2)


def kernel(h, vertices, hadj, pretrained_emb, w_gat, a_src, a_trg, W1, W2, M, Wfc, bfc):
    vidx = vertices[:N_USER].astype(jnp.int32).reshape(N_USER // 128, 128)
    emb = _emb_gather(pretrained_emb, vidx)
    emb_full = jnp.concatenate(
        [emb, jnp.zeros((N - N_USER, F), jnp.float32)], axis=0)
    hf = jnp.concatenate([h, emb_full], axis=1)

    w_cat = jnp.transpose(w_gat, (0, 2, 1, 3)).reshape(K, D, H * F)
    A_s = jnp.zeros((K, H * F, 8), jnp.float32)
    A_t = jnp.zeros((K, H * F, 8), jnp.float32)
    for hh in range(H):
        A_s = A_s.at[:, hh * F:(hh + 1) * F, hh].set(a_src[:, hh, :, 0])
        A_t = A_t.at[:, hh * F:(hh + 1) * F, hh].set(a_trg[:, hh, :, 0])

    hp, asrc, atrg = _tc_proj(hf, w_cat, A_s, A_t)
    hph = hp.reshape(K, N, H, F).transpose(2, 0, 1, 3).reshape(H, K * N, F)
    asrc_t = asrc.transpose(0, 2, 1).reshape(K * H, N)
    atrg_t = atrg.transpose(0, 2, 1).reshape(K * H, N)
    src2 = hadj[:, 0, :].astype(jnp.int32)
    trg2 = hadj[:, 1, :].astype(jnp.int32)

    x = _sc_edge(hph, asrc_t, atrg_t, src2, trg2)
    x = x.reshape(K, H, N_USER, F)

    out = _tc_fusion(hf, x, W1, W2, M, Wfc, bfc.reshape(1, K))
    return out


# speedup vs baseline: 35.4208x; 1.5449x over previous
import jax
import jax.numpy as jnp
from jax import lax
from jax.experimental import pallas as pl
from jax.experimental.pallas import tpu as pltpu
from jax.experimental.pallas import tpu_sc as plsc

N = 10000
N_USER = 8192
E = 320000
D = 192
H = 3
F = 64
K = 2

NC = 2
NS = 16
L = 16

CH = 80
EDGES_PER_TILE = E // NS
NCHUNK = EDGES_PER_TILE // CH

ZCH = 80
NZCHUNK = (N + ZCH - 1) // ZCH

NORM_PER_TILE = N_USER // NS
NCH = 64
NNORM = NORM_PER_TILE // NCH

def _sc_mesh():
    return plsc.VectorSubcoreMesh(
        core_axis_name="c", subcore_axis_name="s",
        num_cores=NC, num_subcores=NS)


def _emb_gather_body(pe_hbm, vidx_hbm, out_hbm, idx_v, rows_v, sem):
    c = lax.axis_index("c")
    s = lax.axis_index("s")
    wid = s * NC + c
    pltpu.sync_copy(vidx_hbm.at[pl.ds(wid * 2, 2)], idx_v)
    for j in range(2):
        pltpu.async_copy(pe_hbm.at[idx_v.at[j]], rows_v.at[j], sem).wait()
        pltpu.sync_copy(rows_v.at[j], out_hbm.at[pl.ds(wid * 256 + j * 128, 128)])


def _emb_gather(pe, vidx):
    return pl.kernel(
        _emb_gather_body,
        out_type=jax.ShapeDtypeStruct((N_USER, F), jnp.float32),
        mesh=_sc_mesh(),
        compiler_params=pltpu.CompilerParams(use_tc_tiling_on_sc=False, needs_layout_passes=False),
        scratch_types=[
            pltpu.VMEM((2, 128), jnp.int32),
            pltpu.VMEM((2, 128, F), jnp.float32),
            pltpu.SemaphoreType.DMA,
        ],
    )(pe, vidx)


def _proj_body(hf_ref, w_ref, as_ref, at_ref, hp_ref, asrc_ref, atrg_ref):
    x = hf_ref[...]
    hp = jnp.dot(x, w_ref[0], preferred_element_type=jnp.float32)
    hp_ref[0] = hp
    asrc_ref[0] = jnp.dot(hp, as_ref[0], preferred_element_type=jnp.float32)[:, :H]
    atrg_ref[0] = jnp.dot(hp, at_ref[0], preferred_element_type=jnp.float32)[:, :H]


def _tc_proj(hf, w_cat, A_s, A_t):
    BN = 2000
    return pl.pallas_call(
        _proj_body,
        grid=(K, N // BN),
        in_specs=[
            pl.BlockSpec((BN, D), lambda k, b: (b, 0)),
            pl.BlockSpec((1, D, H * F), lambda k, b: (k, 0, 0)),
            pl.BlockSpec((1, H * F, 8), lambda k, b: (k, 0, 0)),
            pl.BlockSpec((1, H * F, 8), lambda k, b: (k, 0, 0)),
        ],
        out_specs=[
            pl.BlockSpec((1, BN, H * F), lambda k, b: (k, b, 0)),
            pl.BlockSpec((1, BN, H), lambda k, b: (k, b, 0)),
            pl.BlockSpec((1, BN, H), lambda k, b: (k, b, 0)),
        ],
        out_shape=[
            jax.ShapeDtypeStruct((K, N, H * F), jnp.float32),
            jax.ShapeDtypeStruct((K, N, H), jnp.float32),
            jax.ShapeDtypeStruct((K, N, H), jnp.float32),
        ],
    )(hf, w_cat, A_s, A_t)


NCH2 = NCHUNK // 2


def _edge_body(hph_hbm, asrc_hbm, atrg_hbm, src_hbm, trg_hbm, x_hbm,
               asrc_v, atrg_v, sidx_v, tidx_v, upd_v, w_v, rows_v, out_v,
               gsem0, gsem1, ssem0, ssem1, num_sp, den_sp):
    c = lax.axis_index("c")
    s = lax.axis_index("s")

    iota = lax.iota(jnp.int32, L)
    zeros16 = jnp.zeros((L,), jnp.float32)
    rhalf = iota // 8
    chalf = iota % 8
    iota8 = iota * 8
    base_e = s * EDGES_PER_TILE
    cN = c * N
    nbase = s * NORM_PER_TILE
    gsem = (gsem0, gsem1)
    ssem = (ssem0, ssem1)

    for h in range(H):
        def _z_rows(j, carry):
            for cg in range(F // L):
                rows_v[0, j, pl.ds(cg * L, L)] = zeros16
            return carry
        lax.fori_loop(0, ZCH, _z_rows, None)

        for p in range(2):
            def _z_upd(g, carry, p=p):
                plsc.store_scatter(
                    upd_v, [jnp.full((L,), p, jnp.int32), g * 2 + rhalf, chalf],
                    zeros16)
                return carry
            lax.fori_loop(0, CH * 8 // L, _z_upd, None)

        plsc.subcore_barrier()

        for i in range(8):
            m = s + i * NS

            @pl.when(m < NZCHUNK)
            def _():
                pltpu.sync_copy(rows_v.at[0], num_sp.at[pl.ds(m * ZCH, ZCH)])
                if h == 0:
                    pltpu.sync_copy(upd_v.at[0], den_sp.at[pl.ds(m * ZCH, ZCH)])

        pltpu.sync_copy(asrc_hbm.at[c * H + h], asrc_v)
        pltpu.sync_copy(atrg_hbm.at[c * H + h], atrg_v)

        plsc.subcore_barrier()

        def _issue(cc, p):
            off = base_e + cc * CH
            pltpu.sync_copy(src_hbm.at[c, pl.ds(off, CH)], sidx_v.at[p])
            pltpu.sync_copy(trg_hbm.at[c, pl.ds(off, CH)], tidx_v.at[p])
            pfull = jnp.full((L,), p, jnp.int32)
            hfull = jnp.full((L,), h, jnp.int32)
            for g in range(CH // L):
                s16 = sidx_v[p, pl.ds(g * L, L)]
                t16 = tidx_v[p, pl.ds(g * L, L)]
                a_s = plsc.load_gather(asrc_v, [s16])
                a_t = plsc.load_gather(atrg_v, [t16])
                e = a_s + a_t
                e = jnp.where(e > 0.0, e, 0.2 * e)
                x = jnp.exp(e)
                plsc.store_scatter(upd_v, [pfull, iota + g * L, hfull], x)
                plsc.store_scatter(w_v, [pfull, iota8 + g * L * 8], x)
                sidx_v[p, pl.ds(g * L, L)] = s16 + cN
            pltpu.async_copy(hph_hbm.at[h].at[sidx_v.at[p]], rows_v.at[p], gsem[p])

        def _finish(p):
            pltpu.make_async_copy(
                hph_hbm.at[h].at[sidx_v.at[p]], rows_v.at[p], gsem[p]).wait()

            def _scale(j, carry2, p=p):
                wvec = w_v[p, pl.ds(j * 8, L)]
                w0 = wvec[0]
                for cg in range(F // L):
                    rows_v[p, j, pl.ds(cg * L, L)] = (
                        rows_v[p, j, pl.ds(cg * L, L)] * w0)
                return carry2
            lax.fori_loop(0, CH, _scale, None)

            pltpu.async_copy(rows_v.at[p], num_sp.at[tidx_v.at[p]], ssem[p],
                             add=True)
            pltpu.async_copy(upd_v.at[p], den_sp.at[tidx_v.at[p]], ssem[p],
                             add=True)

        def _wait_scat(p):
            pltpu.make_async_copy(
                rows_v.at[p], num_sp.at[tidx_v.at[p]], ssem[p]).wait()
            pltpu.make_async_copy(
                upd_v.at[p], den_sp.at[tidx_v.at[p]], ssem[p]).wait()

        _issue(0, 0)

        def _body(i, carry):
            cc0 = i * 2

            @pl.when(i > 0)
            def _():
                _wait_scat(1)
            _issue(cc0 + 1, 1)
            _finish(0)
            _finish(1)

            @pl.when(i + 1 < NCH2)
            def _():
                _wait_scat(0)
                _issue(cc0 + 2, 0)
            return carry
        lax.fori_loop(0, NCH2, _body, None)

        _wait_scat(0)
        _wait_scat(1)

        plsc.subcore_barrier()

        zfull = jnp.full((L,), 0, jnp.int32)
        hfull = jnp.full((L,), h, jnp.int32)

        def _norm(ncc, carry):
            nb = nbase + ncc * NCH
            pltpu.sync_copy(num_sp.at[pl.ds(nb, NCH)], rows_v.at[0, pl.ds(0, NCH)])
            pltpu.sync_copy(den_sp.at[pl.ds(nb, NCH)], upd_v.at[0, pl.ds(0, NCH)])

            for jg in range(NCH // L):
                n16 = iota + jg * L
                d16 = plsc.load_gather(upd_v, [zfull, n16, hfull])
                r16 = 1.0 / (d16 + 1e-16)
                plsc.store_scatter(w_v, [zfull, iota8 + jg * L * 8], r16)

            def _node(j, carry2):
                rvec = w_v[0, pl.ds(j * 8, L)]
                r0 = rvec[0]
                for cg in range(F // L):
                    out_v[j, pl.ds(cg * L, L)] = rows_v[0, j, pl.ds(cg * L, L)] * r0
                return carry2
            lax.fori_loop(0, NCH, _node, None)

            pltpu.sync_copy(out_v, x_hbm.at[c * H + h, pl.ds(nb, NCH)])
            return carry
        lax.fori_loop(0, NNORM, _norm, None)


def _sc_edge(hph, asrc_t, atrg_t, src2, trg2):
    return pl.kernel(
        _edge_body,
        out_type=jax.ShapeDtypeStruct((K * H, N_USER, F), jnp.float32),
        mesh=_sc_mesh(),
        compiler_params=pltpu.CompilerParams(use_tc_tiling_on_sc=False, needs_layout_passes=False),
        scratch_types=[
            pltpu.VMEM((N,), jnp.float32),
            pltpu.VMEM((N,), jnp.float32),
            pltpu.VMEM((2, CH), jnp.int32),
            pltpu.VMEM((2, CH), jnp.int32),
            pltpu.VMEM((2, CH, 8), jnp.float32),
            pltpu.VMEM((2, CH * 8 + L), jnp.float32),
            pltpu.VMEM((2, CH, F), jnp.float32),
            pltpu.VMEM((NCH, F), jnp.float32),
            pltpu.SemaphoreType.DMA,
            pltpu.SemaphoreType.DMA,
            pltpu.SemaphoreType.DMA,
            pltpu.SemaphoreType.DMA,
            pltpu.VMEM_SHARED((N, F), jnp.float32),
            pltpu.VMEM_SHARED((N, 8), jnp.float32),
        ],
    )(hph, asrc_t, atrg_t, src2, trg2)


def _fusion_body(hf_ref, x_ref, w1_ref, w2_ref, m_ref, wfc_ref, bfc_ref, out_ref):
    f = hf_ref[...]
    third = jnp.float32(1.0 / 3.0)
    x0 = (x_ref[0, 0] + x_ref[0, 1] + x_ref[0, 2]) * third
    x1 = (x_ref[1, 0] + x_ref[1, 1] + x_ref[1, 2]) * third
    fw1 = jnp.dot(f, w1_ref[...], preferred_element_type=jnp.float32)
    q0 = jnp.tanh(fw1 + jnp.dot(x0, w2_ref[...], preferred_element_type=jnp.float32))
    q1 = jnp.tanh(fw1 + jnp.dot(x1, w2_ref[...], preferred_element_type=jnp.float32))
    s0 = jnp.dot(q0, m_ref[...], preferred_element_type=jnp.float32)
    s1 = jnp.dot(q1, m_ref[...], preferred_element_type=jnp.float32)
    m = jnp.maximum(s0, s1)
    e0 = jnp.exp(s0 - m)
    e1 = jnp.exp(s1 - m)
    denom = e0 + e1
    b0 = e0 / denom
    b1 = e1 / denom
    fus = b0 * x0 + b1 * x1
    wfc = wfc_ref[...]
    logits = (jnp.dot(x0, wfc[0:F], preferred_element_type=jnp.float32)
              + jnp.dot(x1, wfc[F:2 * F], preferred_element_type=jnp.float32)
              + jnp.dot(fus, wfc[2 * F:3 * F], preferred_element_type=jnp.float32)
              + bfc_ref[...])
    lm = jnp.max(logits, axis=-1, keepdims=True)
    lse = lm + jnp.log(jnp.sum(jnp.exp(logits - lm), axis=-1, keepdims=True))
    out_ref[...] = logits - lse


def _tc_fusion(hf, x, W1, W2, M, Wfc, bfc2):
    BN = 512
    return pl.pallas_call(
        _fusion_body,
        grid=(N_USER // BN,),
        in_specs=[
            pl.BlockSpec((BN, D), lambda b: (b, 0)),
            pl.BlockSpec((K, H, BN, F), lambda b: (0, 0, b, 0)),
            pl.BlockSpec((D, F), lambda b: (0, 0)),
            pl.BlockSpec((F, F), lambda b: (0, 0)),
            pl.BlockSpec((F, 1), lambda b: (0, 0)),
            pl.BlockSpec((H * F, K), lambda b: (0, 0)),
            pl.BlockSpec((1, K), lambda b: (0, 0)),
        ],
        out_specs=pl.BlockSpec((BN, K), lambda b: (b, 0)),
        out_shape=jax.ShapeDtypeStruct((N_USER, K), jnp.float32),
    )(hf, x, W1, W2, M, Wfc, bfc2)


def kernel(h, vertices, hadj, pretrained_emb, w_gat, a_src, a_trg, W1, W2, M, Wfc, bfc):
    vidx = vertices[:N_USER].astype(jnp.int32).reshape(N_USER // 128, 128)
    emb = _emb_gather(pretrained_emb, vidx)
    emb_full = jnp.concatenate(
        [emb, jnp.zeros((N - N_USER, F), jnp.float32)], axis=0)
    hf = jnp.concatenate([h, emb_full], axis=1)

    w_cat = jnp.transpose(w_gat, (0, 2, 1, 3)).reshape(K, D, H * F)
    A_s = jnp.zeros((K, H * F, 8), jnp.float32)
    A_t = jnp.zeros((K, H * F, 8), jnp.float32)
    for hh in range(H):
        A_s = A_s.at[:, hh * F:(hh + 1) * F, hh].set(a_src[:, hh, :, 0])
        A_t = A_t.at[:, hh * F:(hh + 1) * F, hh].set(a_trg[:, hh, :, 0])

    hp, asrc, atrg = _tc_proj(hf, w_cat, A_s, A_t)
    hph = hp.reshape(K, N, H, F).transpose(2, 0, 1, 3).reshape(H, K * N, F)
    asrc_t = asrc.transpose(0, 2, 1).reshape(K * H, N)
    atrg_t = atrg.transpose(0, 2, 1).reshape(K * H, N)
    src2 = hadj[:, 0, :].astype(jnp.int32)
    trg2 = hadj[:, 1, :].astype(jnp.int32)

    x = _sc_edge(hph, asrc_t, atrg_t, src2, trg2)
    x = x.reshape(K, H, N_USER, F)

    out = _tc_fusion(hf, x, W1, W2, M, Wfc, bfc.reshape(1, K))
    return out


# speedup vs baseline: 50.9413x; 1.4382x over previous
import jax
import jax.numpy as jnp
from jax import lax
from jax.experimental import pallas as pl
from jax.experimental.pallas import tpu as pltpu
from jax.experimental.pallas import tpu_sc as plsc

N = 10000
N_USER = 8192
E = 320000
D = 192
H = 3
F = 64
K = 2

NC = 2
NS = 16
L = 16

CH = 80
EDGES_PER_TILE = E // NS
NCHUNK = EDGES_PER_TILE // CH

ZCH = 80
NZCHUNK = (N + ZCH - 1) // ZCH

NORM_PER_TILE = N_USER // NS
NCH = 64
NNORM = NORM_PER_TILE // NCH

def _sc_mesh():
    return plsc.VectorSubcoreMesh(
        core_axis_name="c", subcore_axis_name="s",
        num_cores=NC, num_subcores=NS)


def _emb_gather_body(pe_hbm, vidx_hbm, out_hbm, idx_v, rows_v, sem):
    c = lax.axis_index("c")
    s = lax.axis_index("s")
    wid = s * NC + c
    pltpu.sync_copy(vidx_hbm.at[pl.ds(wid * 2, 2)], idx_v)
    for j in range(2):
        pltpu.async_copy(pe_hbm.at[idx_v.at[j]], rows_v.at[j], sem).wait()
        pltpu.sync_copy(rows_v.at[j], out_hbm.at[pl.ds(wid * 256 + j * 128, 128)])


def _emb_gather(pe, vidx):
    return pl.kernel(
        _emb_gather_body,
        out_type=jax.ShapeDtypeStruct((N_USER, F), jnp.float32),
        mesh=_sc_mesh(),
        compiler_params=pltpu.CompilerParams(use_tc_tiling_on_sc=False, needs_layout_passes=False),
        scratch_types=[
            pltpu.VMEM((2, 128), jnp.int32),
            pltpu.VMEM((2, 128, F), jnp.float32),
            pltpu.SemaphoreType.DMA,
        ],
    )(pe, vidx)


def _proj_body(hf_ref, w_ref, as_ref, at_ref, hp_ref, asrc_ref, atrg_ref):
    x = hf_ref[...]
    hp = jnp.dot(x, w_ref[0], preferred_element_type=jnp.float32)
    hp_ref[0] = hp
    asrc_ref[0] = jnp.dot(hp, as_ref[0], preferred_element_type=jnp.float32)[:, :H]
    atrg_ref[0] = jnp.dot(hp, at_ref[0], preferred_element_type=jnp.float32)[:, :H]


def _tc_proj(hf, w_cat, A_s, A_t):
    BN = 2000
    return pl.pallas_call(
        _proj_body,
        grid=(K, N // BN),
        in_specs=[
            pl.BlockSpec((BN, D), lambda k, b: (b, 0)),
            pl.BlockSpec((1, D, H * F), lambda k, b: (k, 0, 0)),
            pl.BlockSpec((1, H * F, 8), lambda k, b: (k, 0, 0)),
            pl.BlockSpec((1, H * F, 8), lambda k, b: (k, 0, 0)),
        ],
        out_specs=[
            pl.BlockSpec((1, BN, H * F), lambda k, b: (k, b, 0)),
            pl.BlockSpec((1, BN, H), lambda k, b: (k, b, 0)),
            pl.BlockSpec((1, BN, H), lambda k, b: (k, b, 0)),
        ],
        out_shape=[
            jax.ShapeDtypeStruct((K, N, H * F), jnp.float32),
            jax.ShapeDtypeStruct((K, N, H), jnp.float32),
            jax.ShapeDtypeStruct((K, N, H), jnp.float32),
        ],
    )(hf, w_cat, A_s, A_t)


NCH2 = NCHUNK // 2


def _edge_body(hph_hbm, asrc_hbm, atrg_hbm, src_hbm, trg_hbm, x_hbm,
               asrc_v, atrg_v, sidx_all, tidx_all, gidx_v, upd_v, w_v, rows_v,
               out_v, gsem0, gsem1, ssem0, ssem1, num_sp, den_sp):
    c = lax.axis_index("c")
    s = lax.axis_index("s")

    iota = lax.iota(jnp.int32, L)
    zeros16 = jnp.zeros((L,), jnp.float32)
    rhalf = iota // 8
    chalf = iota % 8
    iota8 = iota * 8
    base_e = s * EDGES_PER_TILE
    cN = c * N
    nbase = s * NORM_PER_TILE
    gsem = (gsem0, gsem1)
    ssem = (ssem0, ssem1)

    pltpu.sync_copy(src_hbm.at[c, s], sidx_all)
    pltpu.sync_copy(trg_hbm.at[c, s], tidx_all)

    for h in range(H):
        def _z_rows(j, carry):
            for cg in range(F // L):
                rows_v[0, j, pl.ds(cg * L, L)] = zeros16
            return carry
        lax.fori_loop(0, ZCH, _z_rows, None)

        for p in range(2):
            def _z_upd(g, carry, p=p):
                plsc.store_scatter(
                    upd_v, [jnp.full((L,), p, jnp.int32), g * 2 + rhalf, chalf],
                    zeros16)
                return carry
            lax.fori_loop(0, CH * 8 // L, _z_upd, None)

        plsc.subcore_barrier()

        for i in range(8):
            m = s + i * NS

            @pl.when(m < NZCHUNK)
            def _():
                pltpu.sync_copy(rows_v.at[0], num_sp.at[pl.ds(m * ZCH, ZCH)])
                if h == 0:
                    pltpu.sync_copy(upd_v.at[0], den_sp.at[pl.ds(m * ZCH, ZCH)])

        pltpu.sync_copy(asrc_hbm.at[c * H + h], asrc_v)
        pltpu.sync_copy(atrg_hbm.at[c * H + h], atrg_v)

        plsc.subcore_barrier()

        def _issue(cc, p):
            pfull = jnp.full((L,), p, jnp.int32)
            hfull = jnp.full((L,), h, jnp.int32)
            for g in range(CH // L):
                s16 = sidx_all[cc, pl.ds(g * L, L)]
                t16 = tidx_all[cc, pl.ds(g * L, L)]
                a_s = plsc.load_gather(asrc_v, [s16])
                a_t = plsc.load_gather(atrg_v, [t16])
                e = a_s + a_t
                e = jnp.where(e > 0.0, e, 0.2 * e)
                x = jnp.exp(e)
                plsc.store_scatter(upd_v, [pfull, iota + g * L, hfull], x)
                plsc.store_scatter(w_v, [pfull, iota8 + g * L * 8], x)
                gidx_v[p, pl.ds(g * L, L)] = s16 + cN
            pltpu.async_copy(hph_hbm.at[h].at[gidx_v.at[p]], rows_v.at[p], gsem[p])

        def _finish(cc, p):
            pltpu.make_async_copy(
                hph_hbm.at[h].at[gidx_v.at[p]], rows_v.at[p], gsem[p]).wait()

            def _scale(j, carry2, p=p):
                wvec = w_v[p, pl.ds(j * 8, L)]
                w0 = wvec[0]
                for cg in range(F // L):
                    rows_v[p, j, pl.ds(cg * L, L)] = (
                        rows_v[p, j, pl.ds(cg * L, L)] * w0)
                return carry2
            lax.fori_loop(0, CH, _scale, None)

            pltpu.async_copy(rows_v.at[p], num_sp.at[tidx_all.at[cc]], ssem[p],
                             add=True)
            pltpu.async_copy(upd_v.at[p], den_sp.at[tidx_all.at[cc]], ssem[p],
                             add=True)

        def _wait_scat(p):
            pltpu.make_async_copy(
                rows_v.at[p], num_sp.at[tidx_all.at[0]], ssem[p]).wait()
            pltpu.make_async_copy(
                upd_v.at[p], den_sp.at[tidx_all.at[0]], ssem[p]).wait()

        _issue(0, 0)

        def _body(i, carry):
            cc0 = i * 2

            @pl.when(i > 0)
            def _():
                _wait_scat(1)
            _issue(cc0 + 1, 1)
            _finish(cc0, 0)
            _finish(cc0 + 1, 1)

            @pl.when(i + 1 < NCH2)
            def _():
                _wait_scat(0)
                _issue(cc0 + 2, 0)
            return carry
        lax.fori_loop(0, NCH2, _body, None)

        _wait_scat(0)
        _wait_scat(1)

        plsc.subcore_barrier()

        zfull = jnp.full((L,), 0, jnp.int32)
        hfull = jnp.full((L,), h, jnp.int32)

        def _norm(ncc, carry):
            nb = nbase + ncc * NCH
            pltpu.sync_copy(num_sp.at[pl.ds(nb, NCH)], rows_v.at[0, pl.ds(0, NCH)])
            pltpu.sync_copy(den_sp.at[pl.ds(nb, NCH)], upd_v.at[0, pl.ds(0, NCH)])

            for jg in range(NCH // L):
                n16 = iota + jg * L
                d16 = plsc.load_gather(upd_v, [zfull, n16, hfull])
                r16 = 1.0 / (d16 + 1e-16)
                plsc.store_scatter(w_v, [zfull, iota8 + jg * L * 8], r16)

            def _node(j, carry2):
                rvec = w_v[0, pl.ds(j * 8, L)]
                r0 = rvec[0]
                for cg in range(F // L):
                    out_v[j, pl.ds(cg * L, L)] = rows_v[0, j, pl.ds(cg * L, L)] * r0
                return carry2
            lax.fori_loop(0, NCH, _node, None)

            pltpu.sync_copy(out_v, x_hbm.at[c * H + h, pl.ds(nb, NCH)])
            return carry
        lax.fori_loop(0, NNORM, _norm, None)


def _sc_edge(hph, asrc_t, atrg_t, src2, trg2):
    return pl.kernel(
        _edge_body,
        out_type=jax.ShapeDtypeStruct((K * H, N_USER, F), jnp.float32),
        mesh=_sc_mesh(),
        compiler_params=pltpu.CompilerParams(use_tc_tiling_on_sc=False, needs_layout_passes=False),
        scratch_types=[
            pltpu.VMEM((N,), jnp.float32),
            pltpu.VMEM((N,), jnp.float32),
            pltpu.VMEM((NCHUNK, CH), jnp.int32),
            pltpu.VMEM((NCHUNK, CH), jnp.int32),
            pltpu.VMEM((2, CH), jnp.int32),
            pltpu.VMEM((2, CH, 8), jnp.float32),
            pltpu.VMEM((2, CH * 8 + L), jnp.float32),
            pltpu.VMEM((2, CH, F), jnp.float32),
            pltpu.VMEM((NCH, F), jnp.float32),
            pltpu.SemaphoreType.DMA,
            pltpu.SemaphoreType.DMA,
            pltpu.SemaphoreType.DMA,
            pltpu.SemaphoreType.DMA,
            pltpu.VMEM_SHARED((N, F), jnp.float32),
            pltpu.VMEM_SHARED((N, 8), jnp.float32),
        ],
    )(hph, asrc_t, atrg_t, src2, trg2)


def _fusion_body(hf_ref, x_ref, w1_ref, w2_ref, m_ref, wfc_ref, bfc_ref, out_ref):
    f = hf_ref[...]
    third = jnp.float32(1.0 / 3.0)
    x0 = (x_ref[0, 0] + x_ref[0, 1] + x_ref[0, 2]) * third
    x1 = (x_ref[1, 0] + x_ref[1, 1] + x_ref[1, 2]) * third
    fw1 = jnp.dot(f, w1_ref[...], preferred_element_type=jnp.float32)
    q0 = jnp.tanh(fw1 + jnp.dot(x0, w2_ref[...], preferred_element_type=jnp.float32))
    q1 = jnp.tanh(fw1 + jnp.dot(x1, w2_ref[...], preferred_element_type=jnp.float32))
    s0 = jnp.dot(q0, m_ref[...], preferred_element_type=jnp.float32)
    s1 = jnp.dot(q1, m_ref[...], preferred_element_type=jnp.float32)
    m = jnp.maximum(s0, s1)
    e0 = jnp.exp(s0 - m)
    e1 = jnp.exp(s1 - m)
    denom = e0 + e1
    b0 = e0 / denom
    b1 = e1 / denom
    fus = b0 * x0 + b1 * x1
    wfc = wfc_ref[...]
    logits = (jnp.dot(x0, wfc[0:F], preferred_element_type=jnp.float32)
              + jnp.dot(x1, wfc[F:2 * F], preferred_element_type=jnp.float32)
              + jnp.dot(fus, wfc[2 * F:3 * F], preferred_element_type=jnp.float32)
              + bfc_ref[...])
    lm = jnp.max(logits, axis=-1, keepdims=True)
    lse = lm + jnp.log(jnp.sum(jnp.exp(logits - lm), axis=-1, keepdims=True))
    out_ref[...] = logits - lse


def _tc_fusion(hf, x, W1, W2, M, Wfc, bfc2):
    BN = 512
    return pl.pallas_call(
        _fusion_body,
        grid=(N_USER // BN,),
        in_specs=[
            pl.BlockSpec((BN, D), lambda b: (b, 0)),
            pl.BlockSpec((K, H, BN, F), lambda b: (0, 0, b, 0)),
            pl.BlockSpec((D, F), lambda b: (0, 0)),
            pl.BlockSpec((F, F), lambda b: (0, 0)),
            pl.BlockSpec((F, 1), lambda b: (0, 0)),
            pl.BlockSpec((H * F, K), lambda b: (0, 0)),
            pl.BlockSpec((1, K), lambda b: (0, 0)),
        ],
        out_specs=pl.BlockSpec((BN, K), lambda b: (b, 0)),
        out_shape=jax.ShapeDtypeStruct((N_USER, K), jnp.float32),
    )(hf, x, W1, W2, M, Wfc, bfc2)


def kernel(h, vertices, hadj, pretrained_emb, w_gat, a_src, a_trg, W1, W2, M, Wfc, bfc):
    vidx = vertices[:N_USER].astype(jnp.int32).reshape(N_USER // 128, 128)
    emb = _emb_gather(pretrained_emb, vidx)
    emb_full = jnp.concatenate(
        [emb, jnp.zeros((N - N_USER, F), jnp.float32)], axis=0)
    hf = jnp.concatenate([h, emb_full], axis=1)

    w_cat = jnp.transpose(w_gat, (0, 2, 1, 3)).reshape(K, D, H * F)
    A_s = jnp.zeros((K, H * F, 8), jnp.float32)
    A_t = jnp.zeros((K, H * F, 8), jnp.float32)
    for hh in range(H):
        A_s = A_s.at[:, hh * F:(hh + 1) * F, hh].set(a_src[:, hh, :, 0])
        A_t = A_t.at[:, hh * F:(hh + 1) * F, hh].set(a_trg[:, hh, :, 0])

    hp, asrc, atrg = _tc_proj(hf, w_cat, A_s, A_t)
    hph = hp.reshape(K, N, H, F).transpose(2, 0, 1, 3).reshape(H, K * N, F)
    asrc_t = asrc.transpose(0, 2, 1).reshape(K * H, N)
    atrg_t = atrg.transpose(0, 2, 1).reshape(K * H, N)
    src2 = hadj[:, 0, :].astype(jnp.int32).reshape(K, NS, NCHUNK, CH)
    trg2 = hadj[:, 1, :].astype(jnp.int32).reshape(K, NS, NCHUNK, CH)

    x = _sc_edge(hph, asrc_t, atrg_t, src2, trg2)
    x = x.reshape(K, H, N_USER, F)

    out = _tc_fusion(hf, x, W1, W2, M, Wfc, bfc.reshape(1, K))
    return out


# speedup vs baseline: 66.1932x; 1.2994x over previous
import jax
import jax.numpy as jnp
from jax import lax
from jax.experimental import pallas as pl
from jax.experimental.pallas import tpu as pltpu
from jax.experimental.pallas import tpu_sc as plsc

N = 10000
N_USER = 8192
E = 320000
D = 192
H = 3
F = 64
K = 2

NC = 2
NS = 16
L = 16

CH = 80
EDGES_PER_TILE = E // NS
NCHUNK = EDGES_PER_TILE // CH

ZCH = 80
NZCHUNK = (N + ZCH - 1) // ZCH

NORM_PER_TILE = N_USER // NS
NCH = 64
NNORM = NORM_PER_TILE // NCH

def _sc_mesh():
    return plsc.VectorSubcoreMesh(
        core_axis_name="c", subcore_axis_name="s",
        num_cores=NC, num_subcores=NS)


def _emb_gather_body(pe_hbm, vidx_hbm, out_hbm, idx_v, rows_v, sem):
    c = lax.axis_index("c")
    s = lax.axis_index("s")
    wid = s * NC + c
    pltpu.sync_copy(vidx_hbm.at[pl.ds(wid * 2, 2)], idx_v)
    for j in range(2):
        pltpu.async_copy(pe_hbm.at[idx_v.at[j]], rows_v.at[j], sem).wait()
        pltpu.sync_copy(rows_v.at[j], out_hbm.at[pl.ds(wid * 256 + j * 128, 128)])


def _emb_gather(pe, vidx):
    return pl.kernel(
        _emb_gather_body,
        out_type=jax.ShapeDtypeStruct((N_USER, F), jnp.float32),
        mesh=_sc_mesh(),
        compiler_params=pltpu.CompilerParams(use_tc_tiling_on_sc=False, needs_layout_passes=False),
        scratch_types=[
            pltpu.VMEM((2, 128), jnp.int32),
            pltpu.VMEM((2, 128, F), jnp.float32),
            pltpu.SemaphoreType.DMA,
        ],
    )(pe, vidx)


def _proj_body(hf_ref, w_ref, as_ref, at_ref, hp_ref, asrc_ref, atrg_ref):
    x = hf_ref[...]
    hp = jnp.dot(x, w_ref[0], preferred_element_type=jnp.float32)
    hp_ref[0] = hp
    asrc_ref[0] = jnp.dot(hp, as_ref[0], preferred_element_type=jnp.float32)[:, :H]
    atrg_ref[0] = jnp.dot(hp, at_ref[0], preferred_element_type=jnp.float32)[:, :H]


def _tc_proj(hf, w_cat, A_s, A_t):
    BN = 2000
    return pl.pallas_call(
        _proj_body,
        grid=(K, N // BN),
        in_specs=[
            pl.BlockSpec((BN, D), lambda k, b: (b, 0)),
            pl.BlockSpec((1, D, H * F), lambda k, b: (k, 0, 0)),
            pl.BlockSpec((1, H * F, 8), lambda k, b: (k, 0, 0)),
            pl.BlockSpec((1, H * F, 8), lambda k, b: (k, 0, 0)),
        ],
        out_specs=[
            pl.BlockSpec((1, BN, H * F), lambda k, b: (k, b, 0)),
            pl.BlockSpec((1, BN, H), lambda k, b: (k, b, 0)),
            pl.BlockSpec((1, BN, H), lambda k, b: (k, b, 0)),
        ],
        out_shape=[
            jax.ShapeDtypeStruct((K, N, H * F), jnp.float32),
            jax.ShapeDtypeStruct((K, N, H), jnp.float32),
            jax.ShapeDtypeStruct((K, N, H), jnp.float32),
        ],
    )(hf, w_cat, A_s, A_t)


NCH2 = NCHUNK // 2


def _edge_body(hph_hbm, asrc_hbm, atrg_hbm, src_hbm, trg_hbm, x_hbm,
               asrc_v, atrg_v, sidx_all, tidx_all, gidx_v, upd_v, w_v, rows_v,
               out_v, gsem0, gsem1, ssem0, ssem1, num_sp, den_sp):
    c = lax.axis_index("c")
    s = lax.axis_index("s")

    iota = lax.iota(jnp.int32, L)
    zeros16 = jnp.zeros((L,), jnp.float32)
    rhalf = iota // 8
    chalf = iota % 8
    iota8 = iota * 8
    base_e = s * EDGES_PER_TILE
    cN = c * N
    nbase = s * NORM_PER_TILE
    gsem = (gsem0, gsem1)
    ssem = (ssem0, ssem1)

    pltpu.sync_copy(src_hbm.at[c, s], sidx_all)
    pltpu.sync_copy(trg_hbm.at[c, s], tidx_all)

    for h in range(H):
        def _z_rows(j, carry):
            for cg in range(F // L):
                rows_v[0, j, pl.ds(cg * L, L)] = zeros16
            return carry
        lax.fori_loop(0, ZCH, _z_rows, None)

        for p in range(2):
            def _z_upd(g, carry, p=p):
                plsc.store_scatter(
                    upd_v, [jnp.full((L,), p, jnp.int32), g * 2 + rhalf, chalf],
                    zeros16)
                return carry
            lax.fori_loop(0, CH * 8 // L, _z_upd, None)

        plsc.subcore_barrier()

        for i in range(8):
            m = s + i * NS

            @pl.when(m < NZCHUNK)
            def _():
                pltpu.sync_copy(rows_v.at[0], num_sp.at[pl.ds(m * ZCH, ZCH)])
                if h == 0:
                    pltpu.sync_copy(upd_v.at[0], den_sp.at[pl.ds(m * ZCH, ZCH)])

        pltpu.sync_copy(asrc_hbm.at[c * H + h], asrc_v)
        pltpu.sync_copy(atrg_hbm.at[c * H + h], atrg_v)

        plsc.subcore_barrier()

        def _issue(cc, p):
            pfull = jnp.full((L,), p, jnp.int32)
            hfull = jnp.full((L,), h, jnp.int32)
            for g in range(CH // L):
                s16 = sidx_all[cc, pl.ds(g * L, L)]
                t16 = tidx_all[cc, pl.ds(g * L, L)]
                a_s = plsc.load_gather(asrc_v, [s16])
                a_t = plsc.load_gather(atrg_v, [t16])
                e = a_s + a_t
                e = jnp.where(e > 0.0, e, 0.2 * e)
                x = jnp.exp(e)
                plsc.store_scatter(upd_v, [pfull, iota + g * L, hfull], x)
                plsc.store_scatter(w_v, [pfull, iota8 + g * L * 8], x)
                gidx_v[p, pl.ds(g * L, L)] = s16 + cN
            pltpu.async_copy(hph_hbm.at[h].at[gidx_v.at[p]], rows_v.at[p], gsem[p])

        def _finish(cc, p):
            pltpu.make_async_copy(
                hph_hbm.at[h].at[gidx_v.at[p]], rows_v.at[p], gsem[p]).wait()

            @plsc.parallel_loop(0, CH, 1, unroll=4)
            def _scale(j, p=p):
                wvec = w_v[p, pl.ds(j * 8, L)]
                w0 = wvec[0]
                for cg in range(F // L):
                    rows_v[p, j, pl.ds(cg * L, L)] = (
                        rows_v[p, j, pl.ds(cg * L, L)] * w0)

            pltpu.async_copy(rows_v.at[p], num_sp.at[tidx_all.at[cc]], ssem[p],
                             add=True)
            pltpu.async_copy(upd_v.at[p], den_sp.at[tidx_all.at[cc]], ssem[p],
                             add=True)

        def _wait_scat(p):
            pltpu.make_async_copy(
                rows_v.at[p], num_sp.at[tidx_all.at[0]], ssem[p]).wait()
            pltpu.make_async_copy(
                upd_v.at[p], den_sp.at[tidx_all.at[0]], ssem[p]).wait()

        _issue(0, 0)

        def _body(i, carry):
            cc0 = i * 2

            @pl.when(i > 0)
            def _():
                _wait_scat(1)
            _issue(cc0 + 1, 1)
            _finish(cc0, 0)
            _finish(cc0 + 1, 1)

            @pl.when(i + 1 < NCH2)
            def _():
                _wait_scat(0)
                _issue(cc0 + 2, 0)
            return carry
        lax.fori_loop(0, NCH2, _body, None)

        _wait_scat(0)
        _wait_scat(1)

        plsc.subcore_barrier()

        zfull = jnp.full((L,), 0, jnp.int32)
        hfull = jnp.full((L,), h, jnp.int32)

        def _norm(ncc, carry):
            nb = nbase + ncc * NCH
            pltpu.sync_copy(num_sp.at[pl.ds(nb, NCH)], rows_v.at[0, pl.ds(0, NCH)])
            pltpu.sync_copy(den_sp.at[pl.ds(nb, NCH)], upd_v.at[0, pl.ds(0, NCH)])

            for jg in range(NCH // L):
                n16 = iota + jg * L
                d16 = plsc.load_gather(upd_v, [zfull, n16, hfull])
                r16 = 1.0 / (d16 + 1e-16)
                plsc.store_scatter(w_v, [zfull, iota8 + jg * L * 8], r16)

            @plsc.parallel_loop(0, NCH, 1, unroll=4)
            def _node(j):
                rvec = w_v[0, pl.ds(j * 8, L)]
                r0 = rvec[0]
                for cg in range(F // L):
                    out_v[j, pl.ds(cg * L, L)] = rows_v[0, j, pl.ds(cg * L, L)] * r0

            pltpu.sync_copy(out_v, x_hbm.at[c * H + h, pl.ds(nb, NCH)])
            return carry
        lax.fori_loop(0, NNORM, _norm, None)


def _sc_edge(hph, asrc_t, atrg_t, src2, trg2):
    return pl.kernel(
        _edge_body,
        out_type=jax.ShapeDtypeStruct((K * H, N_USER, F), jnp.float32),
        mesh=_sc_mesh(),
        compiler_params=pltpu.CompilerParams(use_tc_tiling_on_sc=False, needs_layout_passes=False),
        scratch_types=[
            pltpu.VMEM((N,), jnp.float32),
            pltpu.VMEM((N,), jnp.float32),
            pltpu.VMEM((NCHUNK, CH), jnp.int32),
            pltpu.VMEM((NCHUNK, CH), jnp.int32),
            pltpu.VMEM((2, CH), jnp.int32),
            pltpu.VMEM((2, CH, 8), jnp.float32),
            pltpu.VMEM((2, CH * 8 + L), jnp.float32),
            pltpu.VMEM((2, CH, F), jnp.float32),
            pltpu.VMEM((NCH, F), jnp.float32),
            pltpu.SemaphoreType.DMA,
            pltpu.SemaphoreType.DMA,
            pltpu.SemaphoreType.DMA,
            pltpu.SemaphoreType.DMA,
            pltpu.VMEM_SHARED((N, F), jnp.float32),
            pltpu.VMEM_SHARED((N, 8), jnp.float32),
        ],
    )(hph, asrc_t, atrg_t, src2, trg2)


def _fusion_body(hf_ref, x_ref, w1_ref, w2_ref, m_ref, wfc_ref, bfc_ref, out_ref):
    f = hf_ref[...]
    third = jnp.float32(1.0 / 3.0)
    x0 = (x_ref[0, 0] + x_ref[0, 1] + x_ref[0, 2]) * third
    x1 = (x_ref[1, 0] + x_ref[1, 1] + x_ref[1, 2]) * third
    fw1 = jnp.dot(f, w1_ref[...], preferred_element_type=jnp.float32)
    q0 = jnp.tanh(fw1 + jnp.dot(x0, w2_ref[...], preferred_element_type=jnp.float32))
    q1 = jnp.tanh(fw1 + jnp.dot(x1, w2_ref[...], preferred_element_type=jnp.float32))
    s0 = jnp.dot(q0, m_ref[...], preferred_element_type=jnp.float32)
    s1 = jnp.dot(q1, m_ref[...], preferred_element_type=jnp.float32)
    m = jnp.maximum(s0, s1)
    e0 = jnp.exp(s0 - m)
    e1 = jnp.exp(s1 - m)
    denom = e0 + e1
    b0 = e0 / denom
    b1 = e1 / denom
    fus = b0 * x0 + b1 * x1
    wfc = wfc_ref[...]
    logits = (jnp.dot(x0, wfc[0:F], preferred_element_type=jnp.float32)
              + jnp.dot(x1, wfc[F:2 * F], preferred_element_type=jnp.float32)
              + jnp.dot(fus, wfc[2 * F:3 * F], preferred_element_type=jnp.float32)
              + bfc_ref[...])
    lm = jnp.max(logits, axis=-1, keepdims=True)
    lse = lm + jnp.log(jnp.sum(jnp.exp(logits - lm), axis=-1, keepdims=True))
    out_ref[...] = logits - lse


def _tc_fusion(hf, x, W1, W2, M, Wfc, bfc2):
    BN = 512
    return pl.pallas_call(
        _fusion_body,
        grid=(N_USER // BN,),
        in_specs=[
            pl.BlockSpec((BN, D), lambda b: (b, 0)),
            pl.BlockSpec((K, H, BN, F), lambda b: (0, 0, b, 0)),
            pl.BlockSpec((D, F), lambda b: (0, 0)),
            pl.BlockSpec((F, F), lambda b: (0, 0)),
            pl.BlockSpec((F, 1), lambda b: (0, 0)),
            pl.BlockSpec((H * F, K), lambda b: (0, 0)),
            pl.BlockSpec((1, K), lambda b: (0, 0)),
        ],
        out_specs=pl.BlockSpec((BN, K), lambda b: (b, 0)),
        out_shape=jax.ShapeDtypeStruct((N_USER, K), jnp.float32),
    )(hf, x, W1, W2, M, Wfc, bfc2)


def kernel(h, vertices, hadj, pretrained_emb, w_gat, a_src, a_trg, W1, W2, M, Wfc, bfc):
    vidx = vertices[:N_USER].astype(jnp.int32).reshape(N_USER // 128, 128)
    emb = _emb_gather(pretrained_emb, vidx)
    emb_full = jnp.concatenate(
        [emb, jnp.zeros((N - N_USER, F), jnp.float32)], axis=0)
    hf = jnp.concatenate([h, emb_full], axis=1)

    w_cat = jnp.transpose(w_gat, (0, 2, 1, 3)).reshape(K, D, H * F)
    A_s = jnp.zeros((K, H * F, 8), jnp.float32)
    A_t = jnp.zeros((K, H * F, 8), jnp.float32)
    for hh in range(H):
        A_s = A_s.at[:, hh * F:(hh + 1) * F, hh].set(a_src[:, hh, :, 0])
        A_t = A_t.at[:, hh * F:(hh + 1) * F, hh].set(a_trg[:, hh, :, 0])

    hp, asrc, atrg = _tc_proj(hf, w_cat, A_s, A_t)
    hph = hp.reshape(K, N, H, F).transpose(2, 0, 1, 3).reshape(H, K * N, F)
    asrc_t = asrc.transpose(0, 2, 1).reshape(K * H, N)
    atrg_t = atrg.transpose(0, 2, 1).reshape(K * H, N)
    src2 = hadj[:, 0, :].astype(jnp.int32).reshape(K, NS, NCHUNK, CH)
    trg2 = hadj[:, 1, :].astype(jnp.int32).reshape(K, NS, NCHUNK, CH)

    x = _sc_edge(hph, asrc_t, atrg_t, src2, trg2)
    x = x.reshape(K, H, N_USER, F)

    out = _tc_fusion(hf, x, W1, W2, M, Wfc, bfc.reshape(1, K))
    return out


# speedup vs baseline: 69.2397x; 1.0460x over previous
import jax
import jax.numpy as jnp
from jax import lax
from jax.experimental import pallas as pl
from jax.experimental.pallas import tpu as pltpu
from jax.experimental.pallas import tpu_sc as plsc

N = 10000
N_USER = 8192
E = 320000
D = 192
H = 3
F = 64
K = 2

NC = 2
NS = 16
L = 16

CH = 80
EDGES_PER_TILE = E // NS
NCHUNK = EDGES_PER_TILE // CH

ZCH = 80
NZCHUNK = (N + ZCH - 1) // ZCH

NORM_PER_TILE = N_USER // NS
NCH = 64
NNORM = NORM_PER_TILE // NCH

def _sc_mesh():
    return plsc.VectorSubcoreMesh(
        core_axis_name="c", subcore_axis_name="s",
        num_cores=NC, num_subcores=NS)


def _emb_gather_body(pe_hbm, vidx_hbm, out_hbm, idx_v, rows_v, sem):
    c = lax.axis_index("c")
    s = lax.axis_index("s")
    wid = s * NC + c
    pltpu.sync_copy(vidx_hbm.at[pl.ds(wid * 2, 2)], idx_v)
    for j in range(2):
        pltpu.async_copy(pe_hbm.at[idx_v.at[j]], rows_v.at[j], sem).wait()
        pltpu.sync_copy(rows_v.at[j], out_hbm.at[pl.ds(wid * 256 + j * 128, 128)])


def _emb_gather(pe, vidx):
    return pl.kernel(
        _emb_gather_body,
        out_type=jax.ShapeDtypeStruct((N_USER, F), jnp.float32),
        mesh=_sc_mesh(),
        compiler_params=pltpu.CompilerParams(use_tc_tiling_on_sc=False, needs_layout_passes=False),
        scratch_types=[
            pltpu.VMEM((2, 128), jnp.int32),
            pltpu.VMEM((2, 128, F), jnp.float32),
            pltpu.SemaphoreType.DMA,
        ],
    )(pe, vidx)


def _proj_body(hf_ref, w_ref, as_ref, at_ref, hp_ref, asrc_ref, atrg_ref):
    x = hf_ref[...]
    hp = jnp.dot(x, w_ref[0], preferred_element_type=jnp.float32)
    bn = hp.shape[0]
    hp_ref[:, 0] = hp.reshape(bn, H, F).transpose(1, 0, 2)
    asrc_ref[0] = jnp.dot(hp, as_ref[0], preferred_element_type=jnp.float32)[:, :H]
    atrg_ref[0] = jnp.dot(hp, at_ref[0], preferred_element_type=jnp.float32)[:, :H]


def _tc_proj(hf, w_cat, A_s, A_t):
    BN = 2000
    return pl.pallas_call(
        _proj_body,
        grid=(K, N // BN),
        in_specs=[
            pl.BlockSpec((BN, D), lambda k, b: (b, 0)),
            pl.BlockSpec((1, D, H * F), lambda k, b: (k, 0, 0)),
            pl.BlockSpec((1, H * F, 8), lambda k, b: (k, 0, 0)),
            pl.BlockSpec((1, H * F, 8), lambda k, b: (k, 0, 0)),
        ],
        out_specs=[
            pl.BlockSpec((H, 1, BN, F), lambda k, b: (0, k, b, 0)),
            pl.BlockSpec((1, BN, H), lambda k, b: (k, b, 0)),
            pl.BlockSpec((1, BN, H), lambda k, b: (k, b, 0)),
        ],
        out_shape=[
            jax.ShapeDtypeStruct((H, K, N, F), jnp.float32),
            jax.ShapeDtypeStruct((K, N, H), jnp.float32),
            jax.ShapeDtypeStruct((K, N, H), jnp.float32),
        ],
    )(hf, w_cat, A_s, A_t)


NCH2 = NCHUNK // 2


def _edge_body(hph_hbm, asrc_hbm, atrg_hbm, src_hbm, trg_hbm, x_hbm,
               asrc_v, atrg_v, sidx_all, tidx_all, gidx_v, upd_v, w_v, rows_v,
               out_v, gsem0, gsem1, ssem0, ssem1, num_sp, den_sp):
    c = lax.axis_index("c")
    s = lax.axis_index("s")

    iota = lax.iota(jnp.int32, L)
    zeros16 = jnp.zeros((L,), jnp.float32)
    rhalf = iota // 8
    chalf = iota % 8
    iota8 = iota * 8
    base_e = s * EDGES_PER_TILE
    cN = c * N
    nbase = s * NORM_PER_TILE
    gsem = (gsem0, gsem1)
    ssem = (ssem0, ssem1)

    pltpu.sync_copy(src_hbm.at[c, s], sidx_all)
    pltpu.sync_copy(trg_hbm.at[c, s], tidx_all)

    for h in range(H):
        def _z_rows(j, carry):
            for cg in range(F // L):
                rows_v[0, j, pl.ds(cg * L, L)] = zeros16
            return carry
        lax.fori_loop(0, ZCH, _z_rows, None)

        for p in range(2):
            def _z_upd(g, carry, p=p):
                plsc.store_scatter(
                    upd_v, [jnp.full((L,), p, jnp.int32), g * 2 + rhalf, chalf],
                    zeros16)
                return carry
            lax.fori_loop(0, CH * 8 // L, _z_upd, None)

        plsc.subcore_barrier()

        for i in range(8):
            m = s + i * NS

            @pl.when(m < NZCHUNK)
            def _():
                pltpu.sync_copy(rows_v.at[0], num_sp.at[pl.ds(m * ZCH, ZCH)])
                if h == 0:
                    pltpu.sync_copy(upd_v.at[0], den_sp.at[pl.ds(m * ZCH, ZCH)])

        pltpu.sync_copy(asrc_hbm.at[c * H + h], asrc_v)
        pltpu.sync_copy(atrg_hbm.at[c * H + h], atrg_v)

        plsc.subcore_barrier()

        def _issue(cc, p):
            pfull = jnp.full((L,), p, jnp.int32)
            hfull = jnp.full((L,), h, jnp.int32)
            for g in range(CH // L):
                s16 = sidx_all[cc, pl.ds(g * L, L)]
                t16 = tidx_all[cc, pl.ds(g * L, L)]
                a_s = plsc.load_gather(asrc_v, [s16])
                a_t = plsc.load_gather(atrg_v, [t16])
                e = a_s + a_t
                e = jnp.where(e > 0.0, e, 0.2 * e)
                x = jnp.exp(e)
                plsc.store_scatter(upd_v, [pfull, iota + g * L, hfull], x)
                plsc.store_scatter(w_v, [pfull, iota8 + g * L * 8], x)
                gidx_v[p, pl.ds(g * L, L)] = s16 + cN
            pltpu.async_copy(hph_hbm.at[h].at[gidx_v.at[p]], rows_v.at[p], gsem[p])

        def _finish(cc, p):
            pltpu.make_async_copy(
                hph_hbm.at[h].at[gidx_v.at[p]], rows_v.at[p], gsem[p]).wait()

            @plsc.parallel_loop(0, CH, 1, unroll=8)
            def _scale(j, p=p):
                wvec = w_v[p, pl.ds(j * 8, L)]
                w0 = wvec[0]
                for cg in range(F // L):
                    rows_v[p, j, pl.ds(cg * L, L)] = (
                        rows_v[p, j, pl.ds(cg * L, L)] * w0)

            pltpu.async_copy(rows_v.at[p], num_sp.at[tidx_all.at[cc]], ssem[p],
                             add=True)
            pltpu.async_copy(upd_v.at[p], den_sp.at[tidx_all.at[cc]], ssem[p],
                             add=True)

        def _wait_scat(p):
            pltpu.make_async_copy(
                rows_v.at[p], num_sp.at[tidx_all.at[0]], ssem[p]).wait()
            pltpu.make_async_copy(
                upd_v.at[p], den_sp.at[tidx_all.at[0]], ssem[p]).wait()

        _issue(0, 0)

        def _body(i, carry):
            cc0 = i * 2

            @pl.when(i > 0)
            def _():
                _wait_scat(1)
            _issue(cc0 + 1, 1)
            _finish(cc0, 0)
            _finish(cc0 + 1, 1)

            @pl.when(i + 1 < NCH2)
            def _():
                _wait_scat(0)
                _issue(cc0 + 2, 0)
            return carry
        lax.fori_loop(0, NCH2, _body, None)

        _wait_scat(0)
        _wait_scat(1)

        plsc.subcore_barrier()

        zfull = jnp.full((L,), 0, jnp.int32)
        hfull = jnp.full((L,), h, jnp.int32)

        def _norm(ncc, carry):
            nb = nbase + ncc * NCH
            pltpu.sync_copy(num_sp.at[pl.ds(nb, NCH)], rows_v.at[0, pl.ds(0, NCH)])
            pltpu.sync_copy(den_sp.at[pl.ds(nb, NCH)], upd_v.at[0, pl.ds(0, NCH)])

            for jg in range(NCH // L):
                n16 = iota + jg * L
                d16 = plsc.load_gather(upd_v, [zfull, n16, hfull])
                r16 = 1.0 / (d16 + 1e-16)
                plsc.store_scatter(w_v, [zfull, iota8 + jg * L * 8], r16)

            @plsc.parallel_loop(0, NCH, 1, unroll=4)
            def _node(j):
                rvec = w_v[0, pl.ds(j * 8, L)]
                r0 = rvec[0]
                for cg in range(F // L):
                    out_v[j, pl.ds(cg * L, L)] = rows_v[0, j, pl.ds(cg * L, L)] * r0

            pltpu.sync_copy(out_v, x_hbm.at[c * H + h, pl.ds(nb, NCH)])
            return carry
        lax.fori_loop(0, NNORM, _norm, None)


def _sc_edge(hph, asrc_t, atrg_t, src2, trg2):
    return pl.kernel(
        _edge_body,
        out_type=jax.ShapeDtypeStruct((K * H, N_USER, F), jnp.float32),
        mesh=_sc_mesh(),
        compiler_params=pltpu.CompilerParams(use_tc_tiling_on_sc=False, needs_layout_passes=False),
        scratch_types=[
            pltpu.VMEM((N,), jnp.float32),
            pltpu.VMEM((N,), jnp.float32),
            pltpu.VMEM((NCHUNK, CH), jnp.int32),
            pltpu.VMEM((NCHUNK, CH), jnp.int32),
            pltpu.VMEM((2, CH), jnp.int32),
            pltpu.VMEM((2, CH, 8), jnp.float32),
            pltpu.VMEM((2, CH * 8 + L), jnp.float32),
            pltpu.VMEM((2, CH, F), jnp.float32),
            pltpu.VMEM((NCH, F), jnp.float32),
            pltpu.SemaphoreType.DMA,
            pltpu.SemaphoreType.DMA,
            pltpu.SemaphoreType.DMA,
            pltpu.SemaphoreType.DMA,
            pltpu.VMEM_SHARED((N, F), jnp.float32),
            pltpu.VMEM_SHARED((N, 8), jnp.float32),
        ],
    )(hph, asrc_t, atrg_t, src2, trg2)


def _fusion_body(hf_ref, x_ref, w1_ref, w2_ref, m_ref, wfc_ref, bfc_ref, out_ref):
    f = hf_ref[...]
    third = jnp.float32(1.0 / 3.0)
    x0 = (x_ref[0, 0] + x_ref[0, 1] + x_ref[0, 2]) * third
    x1 = (x_ref[1, 0] + x_ref[1, 1] + x_ref[1, 2]) * third
    fw1 = jnp.dot(f, w1_ref[...], preferred_element_type=jnp.float32)
    q0 = jnp.tanh(fw1 + jnp.dot(x0, w2_ref[...], preferred_element_type=jnp.float32))
    q1 = jnp.tanh(fw1 + jnp.dot(x1, w2_ref[...], preferred_element_type=jnp.float32))
    s0 = jnp.dot(q0, m_ref[...], preferred_element_type=jnp.float32)
    s1 = jnp.dot(q1, m_ref[...], preferred_element_type=jnp.float32)
    m = jnp.maximum(s0, s1)
    e0 = jnp.exp(s0 - m)
    e1 = jnp.exp(s1 - m)
    denom = e0 + e1
    b0 = e0 / denom
    b1 = e1 / denom
    fus = b0 * x0 + b1 * x1
    wfc = wfc_ref[...]
    logits = (jnp.dot(x0, wfc[0:F], preferred_element_type=jnp.float32)
              + jnp.dot(x1, wfc[F:2 * F], preferred_element_type=jnp.float32)
              + jnp.dot(fus, wfc[2 * F:3 * F], preferred_element_type=jnp.float32)
              + bfc_ref[...])
    lm = jnp.max(logits, axis=-1, keepdims=True)
    lse = lm + jnp.log(jnp.sum(jnp.exp(logits - lm), axis=-1, keepdims=True))
    out_ref[...] = logits - lse


def _tc_fusion(hf, x, W1, W2, M, Wfc, bfc2):
    BN = 512
    return pl.pallas_call(
        _fusion_body,
        grid=(N_USER // BN,),
        in_specs=[
            pl.BlockSpec((BN, D), lambda b: (b, 0)),
            pl.BlockSpec((K, H, BN, F), lambda b: (0, 0, b, 0)),
            pl.BlockSpec((D, F), lambda b: (0, 0)),
            pl.BlockSpec((F, F), lambda b: (0, 0)),
            pl.BlockSpec((F, 1), lambda b: (0, 0)),
            pl.BlockSpec((H * F, K), lambda b: (0, 0)),
            pl.BlockSpec((1, K), lambda b: (0, 0)),
        ],
        out_specs=pl.BlockSpec((BN, K), lambda b: (b, 0)),
        out_shape=jax.ShapeDtypeStruct((N_USER, K), jnp.float32),
    )(hf, x, W1, W2, M, Wfc, bfc2)


def kernel(h, vertices, hadj, pretrained_emb, w_gat, a_src, a_trg, W1, W2, M, Wfc, bfc):
    vidx = vertices[:N_USER].astype(jnp.int32).reshape(N_USER // 128, 128)
    emb = _emb_gather(pretrained_emb, vidx)
    emb_full = jnp.concatenate(
        [emb, jnp.zeros((N - N_USER, F), jnp.float32)], axis=0)
    hf = jnp.concatenate([h, emb_full], axis=1)

    w_cat = jnp.transpose(w_gat, (0, 2, 1, 3)).reshape(K, D, H * F)
    A_s = jnp.zeros((K, H * F, 8), jnp.float32)
    A_t = jnp.zeros((K, H * F, 8), jnp.float32)
    for hh in range(H):
        A_s = A_s.at[:, hh * F:(hh + 1) * F, hh].set(a_src[:, hh, :, 0])
        A_t = A_t.at[:, hh * F:(hh + 1) * F, hh].set(a_trg[:, hh, :, 0])

    hph4, asrc, atrg = _tc_proj(hf, w_cat, A_s, A_t)
    hph = hph4.reshape(H, K * N, F)
    asrc_t = asrc.transpose(0, 2, 1).reshape(K * H, N)
    atrg_t = atrg.transpose(0, 2, 1).reshape(K * H, N)
    src2 = hadj[:, 0, :].astype(jnp.int32).reshape(K, NS, NCHUNK, CH)
    trg2 = hadj[:, 1, :].astype(jnp.int32).reshape(K, NS, NCHUNK, CH)

    x = _sc_edge(hph, asrc_t, atrg_t, src2, trg2)
    x = x.reshape(K, H, N_USER, F)

    out = _tc_fusion(hf, x, W1, W2, M, Wfc, bfc.reshape(1, K))
    return out
